# baseline (speedup 1.0000x reference)
import jax
import jax.numpy as jnp
from jax.experimental import pallas as pl

_B = 1024
_D = 362
_DPAD = 384
_A = 3
_K = 10000
_KPAD = 10112
_KNN = 50
_BR = 256
_VD = 8

_INF_BITS = 0x7F800000


def _planner_kernel(x_ref, a_ref, kt_ref, vals_ref, out_ref, misc_ref):
    x = x_ref[...]
    col = jax.lax.broadcasted_iota(jnp.int32, (_BR, _DPAD), 1)

    m = jnp.max(x, axis=1, keepdims=True)
    p = jnp.min(jnp.where(x == m, col, _DPAD), axis=1, keepdims=True)
    cells = [
        jnp.zeros_like(p),
        p,
        jnp.clip(p - 19, 1, 361),
        jnp.clip(p + 19, 1, 361),
        jnp.clip(p - 1, 1, 361),
        jnp.clip(p + 1, 1, 361),
    ]
    att = [x[:, 0:1], m]
    for j in range(2, 6):
        att.append(jnp.sum(jnp.where(col == cells[j], x, 0.0), axis=1,
                           keepdims=True))
    q2 = att[0] * att[0]
    for j in range(1, 6):
        q2 = q2 + att[j] * att[j]

    act = a_ref[...]

    attb = [(-2.0 * aj.astype(jnp.bfloat16).astype(jnp.float32))
            for aj in att]
    qcol = jax.lax.broadcasted_iota(jnp.int32, (_BR, _A * 8), 1)
    qrel = qcol - act * 8
    attcat = jnp.zeros((_BR, _A * 8), jnp.float32)
    for d in range(6):
        attcat = jnp.where(qrel == d, attb[d], attcat)
    ktall = kt_ref[...]
    qkm2 = jnp.dot(attcat.astype(jnp.bfloat16), ktall.astype(jnp.bfloat16),
                   preferred_element_type=jnp.float32)

    kk2 = []
    for a in range(_A):
        kta = ktall[8 * a:8 * a + 8]
        kk2.append(jnp.sum(kta * kta, axis=0, keepdims=True))
    kk2_sel = jnp.where(act == 0, kk2[0],
                        jnp.where(act == 1, kk2[1], kk2[2]))
    d2 = (q2 + kk2_sel) + qkm2

    uncert = jnp.min(d2, axis=1, keepdims=True)
    d2c = jnp.maximum(d2, 0.0)
    bits = jax.lax.bitcast_convert_type(d2c, jnp.int32)

    def _count_le(arr_le):
        return jnp.sum(arr_le.astype(jnp.int32), axis=1, keepdims=True)

    def bs_body(_, lohi):
        lo, hi = lohi
        mid = jax.lax.shift_right_logical(lo + hi, 1)
        cnt = _count_le(bits <= mid)
        ge = cnt >= _KNN
        return jnp.where(ge, lo, mid + 1), jnp.where(ge, mid, hi)

    lo0 = jnp.zeros((_BR, 1), jnp.int32)
    hi0 = jnp.full((_BR, 1), _INF_BITS, jnp.int32)
    thr, _ = jax.lax.fori_loop(0, 31, bs_body, (lo0, hi0))

    cle = _count_le(bits <= thr)
    rcpv = 1.0 / (d2c + 1e-3)

    def _wm_ties(_):
        kidx = jax.lax.broadcasted_iota(jnp.int32, (1, _KPAD), 1)
        eq = bits == thr
        nlt = cle - _count_le(eq)

        def js_body(_, lohi):
            lo, hi = lohi
            mid = jax.lax.shift_right_logical(lo + hi, 1)
            cnt = nlt + _count_le(eq & (kidx <= mid))
            ge = cnt >= _KNN
            return jnp.where(ge, lo, mid + 1), jnp.where(ge, mid, hi)

        jlo0 = jnp.zeros((_BR, 1), jnp.int32)
        jhi0 = jnp.full((_BR, 1), _KPAD - 1, jnp.int32)
        j_lo, _ = jax.lax.fori_loop(0, 14, js_body, (jlo0, jhi0))
        return jnp.where((bits < thr) | (eq & (kidx <= j_lo)), rcpv, 0.0)

    def _wm_noties(_):
        return jnp.where(bits <= thr, rcpv, 0.0)

    wm = jax.lax.cond(jnp.any(cle != _KNN), _wm_ties, _wm_noties, 0)
    s = jnp.sum(wm, axis=1, keepdims=True)
    wn = (wm * (1.0 / s)).astype(jnp.bfloat16)
    predall = jnp.dot(wn, vals_ref[...].astype(jnp.bfloat16),
                      preferred_element_type=jnp.float32)
    pred = jnp.where(act == 0, predall[:, 0:_VD],
                     jnp.where(act == 1, predall[:, _VD:2 * _VD],
                               predall[:, 2 * _VD:3 * _VD]))

    tdelta = jnp.zeros((_BR, _DPAD), jnp.float32)
    for j in range(6):
        tdelta = jnp.where(col == cells[j], pred[:, j:j + 1], tdelta)
    out_ref[...] = x + tdelta

    mcol = jax.lax.broadcasted_iota(jnp.int32, (_BR, _VD), 1)
    misc_ref[...] = jnp.where(mcol == 7, uncert, pred)


def kernel(batch_x, batch_a, mem_keys, mem_vals):
    xpad = jnp.pad(batch_x, ((0, 0), (0, _DPAD - _D)),
                   constant_values=-jnp.inf)
    a2d = batch_a.astype(jnp.int32).reshape(_B, 1)
    kt = jnp.transpose(mem_keys, (0, 2, 1))
    kt = jnp.pad(kt, ((0, 0), (0, 0), (0, _KPAD - _K)),
                 constant_values=1e9)
    kt = jnp.pad(kt, ((0, 0), (0, 2), (0, 0)))
    ktall = kt.reshape(_A * 8, _KPAD)
    valsp = jnp.pad(mem_vals, ((0, 0), (0, _KPAD - _K), (0, 1)))
    vcat = jnp.transpose(valsp, (1, 0, 2)).reshape(_KPAD, _A * _VD)

    out, misc = pl.pallas_call(
        _planner_kernel,
        grid=(_B // _BR,),
        in_specs=[
            pl.BlockSpec((_BR, _DPAD), lambda i: (i, 0)),
            pl.BlockSpec((_BR, 1), lambda i: (i, 0)),
            pl.BlockSpec((_A * 8, _KPAD), lambda i: (0, 0)),
            pl.BlockSpec((_KPAD, _A * _VD), lambda i: (0, 0)),
        ],
        out_specs=[
            pl.BlockSpec((_BR, _DPAD), lambda i: (i, 0)),
            pl.BlockSpec((_BR, _VD), lambda i: (i, 0)),
        ],
        out_shape=[
            jax.ShapeDtypeStruct((_B, _DPAD), jnp.float32),
            jax.ShapeDtypeStruct((_B, _VD), jnp.float32),
        ],
    )(xpad, a2d, ktall, vcat)

    pred_next = out[:, :_D]
    uncertainty = misc[:, 7]
    done = misc[:, 6]
    return pred_next, uncertainty, done


# speedup vs baseline: 1.0638x; 1.0638x over previous
import jax
import jax.numpy as jnp
from jax.experimental import pallas as pl

_B = 1024
_D = 362
_DPAD = 384
_A = 3
_K = 10000
_KPAD = 10112
_KNN = 50
_BR = 256
_VD = 8

_INF_BITS = 0x7F800000


def _planner_kernel(x_ref, a_ref, kt_ref, vals_ref, out_ref, misc_ref):
    x = x_ref[...]
    col = jax.lax.broadcasted_iota(jnp.int32, (_BR, _DPAD), 1)

    m = jnp.max(x, axis=1, keepdims=True)
    p = jnp.min(jnp.where(x == m, col, _DPAD), axis=1, keepdims=True)
    cells = [
        jnp.zeros_like(p),
        p,
        jnp.clip(p - 19, 1, 361),
        jnp.clip(p + 19, 1, 361),
        jnp.clip(p - 1, 1, 361),
        jnp.clip(p + 1, 1, 361),
    ]
    att = [x[:, 0:1], m]
    for j in range(2, 6):
        att.append(jnp.sum(jnp.where(col == cells[j], x, 0.0), axis=1,
                           keepdims=True))
    q2 = att[0] * att[0]
    for j in range(1, 6):
        q2 = q2 + att[j] * att[j]

    act = a_ref[...]

    attb = [(-2.0 * aj.astype(jnp.bfloat16).astype(jnp.float32))
            for aj in att]
    qcol = jax.lax.broadcasted_iota(jnp.int32, (_BR, _A * 8), 1)
    qrel = qcol - act * 8
    attcat = jnp.zeros((_BR, _A * 8), jnp.float32)
    for d in range(6):
        attcat = jnp.where(qrel == d, attb[d], attcat)
    ktall = kt_ref[...]
    qkm2 = jnp.dot(attcat.astype(jnp.bfloat16), ktall.astype(jnp.bfloat16),
                   preferred_element_type=jnp.float32)

    kk2 = []
    for a in range(_A):
        kta = ktall[8 * a:8 * a + 8]
        kk2.append(jnp.sum(kta * kta, axis=0, keepdims=True))
    kk2_sel = jnp.where(act == 0, kk2[0],
                        jnp.where(act == 1, kk2[1], kk2[2]))
    d2 = (q2 + kk2_sel) + qkm2

    uncert = jnp.min(d2, axis=1, keepdims=True)
    d2c = jnp.maximum(d2, 0.0)
    bits = jax.lax.bitcast_convert_type(d2c, jnp.int32)

    def _count_le(arr_le):
        return jnp.sum(arr_le.astype(jnp.int32), axis=1, keepdims=True)

    def bs_body(_, lohi):
        lo, hi = lohi
        mid = jax.lax.shift_right_logical(lo + hi, 1)
        cnt = _count_le(bits <= mid)
        ge = cnt >= _KNN
        return jnp.where(ge, lo, mid + 1), jnp.where(ge, mid, hi)

    lo0 = jnp.zeros((_BR, 1), jnp.int32)
    hi0 = jnp.full((_BR, 1), _INF_BITS, jnp.int32)
    thr, _ = jax.lax.fori_loop(0, 31, bs_body, (lo0, hi0))

    cle = _count_le(bits <= thr)
    kidx = jax.lax.broadcasted_iota(jnp.int32, (1, _KPAD), 1)

    eq = bits == thr

    def _tie_search(_):
        nlt = cle - _count_le(eq)

        def js_body(_, lohi):
            lo, hi = lohi
            mid = jax.lax.shift_right_logical(lo + hi, 1)
            cnt = nlt + _count_le(eq & (kidx <= mid))
            ge = cnt >= _KNN
            return jnp.where(ge, lo, mid + 1), jnp.where(ge, mid, hi)

        jlo0 = jnp.zeros((_BR, 1), jnp.int32)
        jhi0 = jnp.full((_BR, 1), _KPAD - 1, jnp.int32)
        j_lo, _ = jax.lax.fori_loop(0, 14, js_body, (jlo0, jhi0))
        return j_lo

    def _no_ties(_):
        return jnp.full((_BR, 1), _KPAD - 1, jnp.int32)

    j_lo = jax.lax.cond(jnp.any(cle != _KNN), _tie_search, _no_ties, 0)

    mask = (bits < thr) | (eq & (kidx <= j_lo))

    wm = jnp.where(mask, 1.0 / (d2c + 1e-3), 0.0)
    s = jnp.sum(wm, axis=1, keepdims=True)
    wn = (wm * (1.0 / s)).astype(jnp.bfloat16)
    predall = jnp.dot(wn, vals_ref[...].astype(jnp.bfloat16),
                      preferred_element_type=jnp.float32)
    pred = jnp.where(act == 0, predall[:, 0:_VD],
                     jnp.where(act == 1, predall[:, _VD:2 * _VD],
                               predall[:, 2 * _VD:3 * _VD]))

    tdelta = jnp.zeros((_BR, _DPAD), jnp.float32)
    for j in range(6):
        tdelta = jnp.where(col == cells[j], pred[:, j:j + 1], tdelta)
    out_ref[...] = x + tdelta

    mcol = jax.lax.broadcasted_iota(jnp.int32, (_BR, _VD), 1)
    misc_ref[...] = jnp.where(mcol == 7, uncert, pred)


def kernel(batch_x, batch_a, mem_keys, mem_vals):
    xpad = jnp.pad(batch_x, ((0, 0), (0, _DPAD - _D)),
                   constant_values=-jnp.inf)
    a2d = batch_a.astype(jnp.int32).reshape(_B, 1)
    kt = jnp.transpose(mem_keys, (0, 2, 1))
    kt = jnp.pad(kt, ((0, 0), (0, 0), (0, _KPAD - _K)),
                 constant_values=1e9)
    kt = jnp.pad(kt, ((0, 0), (0, 2), (0, 0)))
    ktall = kt.reshape(_A * 8, _KPAD)
    valsp = jnp.pad(mem_vals, ((0, 0), (0, _KPAD - _K), (0, 1)))
    vcat = jnp.transpose(valsp, (1, 0, 2)).reshape(_KPAD, _A * _VD)

    out, misc = pl.pallas_call(
        _planner_kernel,
        grid=(_B // _BR,),
        in_specs=[
            pl.BlockSpec((_BR, _DPAD), lambda i: (i, 0)),
            pl.BlockSpec((_BR, 1), lambda i: (i, 0)),
            pl.BlockSpec((_A * 8, _KPAD), lambda i: (0, 0)),
            pl.BlockSpec((_KPAD, _A * _VD), lambda i: (0, 0)),
        ],
        out_specs=[
            pl.BlockSpec((_BR, _DPAD), lambda i: (i, 0)),
            pl.BlockSpec((_BR, _VD), lambda i: (i, 0)),
        ],
        out_shape=[
            jax.ShapeDtypeStruct((_B, _DPAD), jnp.float32),
            jax.ShapeDtypeStruct((_B, _VD), jnp.float32),
        ],
    )(xpad, a2d, ktall, vcat)

    pred_next = out[:, :_D]
    uncertainty = misc[:, 7]
    done = misc[:, 6]
    return pred_next, uncertainty, done
